# Initial kernel scaffold; baseline (speedup 1.0000x reference)
import jax
import jax.numpy as jnp
from jax.experimental import pallas as pl


def kernel(x, edge_index, edge_label, weight, q, k, bias):
    raise NotImplementedError("write your pallas kernel here")


# speedup vs baseline: 3.2549x; 3.2549x over previous
import jax
import jax.numpy as jnp
from jax import lax
from jax.experimental import pallas as pl
from jax.experimental.pallas import tpu as pltpu
from jax.experimental.pallas import tpu_sc as plsc

N = 10000
E = 160000
IN_CH = 128
OUT_CH = 256
HEADS = 2
NUM_REL = 8
NEG_SLOPE = 0.2

NC = 2
NS = 16
NW = NC * NS
EP = 163840
EPW = EP // NW
PADN = EP - E
CH = 128
NCHUNK = EPW // CH
RH = NUM_REL * HEADS

NPAD = 10240
DEN_W = 2 * NPAD
DSTRIPE = DEN_W // NS

RANGE = 640
NPASS = 16
NPADN = NPASS * RANGE
GROUPS = NUM_REL * HEADS
SLAB_USED = GROUPS * RANGE
SLAB_ROWS = 10368
ZSTRIPE = SLAB_ROWS // NS
DUMP_ROW = SLAB_USED
SUB = 32
NSUBW = EPW // SUB

_i32 = jnp.int32
_f32 = jnp.float32


def _iota16():
    return lax.iota(_i32, 16)


def _full16(v):
    return jnp.full((16,), v, _i32)


def _k1_body(x_ref, w_ref, q_ref, k_ref, qt_ref, kt_ref):
    wq = jnp.concatenate(
        [jnp.dot(w_ref[r], q_ref[...], preferred_element_type=_f32)
         for r in range(NUM_REL)], axis=1)
    wk = jnp.concatenate(
        [jnp.dot(w_ref[r], k_ref[...], preferred_element_type=_f32)
         for r in range(NUM_REL)], axis=1)
    xt = x_ref[...]
    qt_ref[...] = jnp.dot(xt, wq, preferred_element_type=_f32)
    kt_ref[...] = jnp.dot(xt, wk, preferred_element_type=_f32)


def _k1(x, weight, q, k):
    return pl.pallas_call(
        _k1_body,
        out_shape=[
            jax.ShapeDtypeStruct((N, RH), _f32),
            jax.ShapeDtypeStruct((N, RH), _f32),
        ],
    )(x, weight, q, k)


def _expa_chunk(qtf, ktf, sbuf, dbuf, rbuf, off, iD0, iD1, iS0, iS1,
                g0, g1, g2, g3, sem, emit, idx_extra=None, extra_fire=None):
    def idx_body(g, _):
        d = dbuf[pl.ds(off + g * 16, 16)]
        r_ = rbuf[pl.ds(off + g * 16, 16)]
        s_ = sbuf[pl.ds(off + g * 16, 16)]
        bD = jnp.minimum(d, N - 1) * RH + r_ * HEADS
        bS = s_ * RH + r_ * HEADS
        iD0[pl.ds(g * 16, 16)] = bD
        iD1[pl.ds(g * 16, 16)] = bD + 1
        iS0[pl.ds(g * 16, 16)] = bS
        iS1[pl.ds(g * 16, 16)] = bS + 1
        if idx_extra is not None:
            idx_extra(g, d)
        return 0

    lax.fori_loop(0, CH // 16, idx_body, 0)

    cps = [
        pltpu.async_copy(qtf.at[iD0], g0, sem),
        pltpu.async_copy(qtf.at[iD1], g1, sem),
        pltpu.async_copy(ktf.at[iS0], g2, sem),
        pltpu.async_copy(ktf.at[iS1], g3, sem),
    ]
    if extra_fire is not None:
        cps.extend(extra_fire())
    for cp in cps:
        cp.wait()

    def grp_body(g, _):
        a0 = g0[pl.ds(g * 16, 16)] + g2[pl.ds(g * 16, 16)]
        a1 = g1[pl.ds(g * 16, 16)] + g3[pl.ds(g * 16, 16)]
        a0 = jnp.where(a0 > 0.0, a0, a0 * NEG_SLOPE)
        a1 = jnp.where(a1 > 0.0, a1, a1 * NEG_SLOPE)
        emit(g, jnp.exp(a0), jnp.exp(a1))
        return 0

    lax.fori_loop(0, CH // 16, grp_body, 0)


def _k2_body(qtf, ktf, srcp, dstp, relp, denout,
             sbuf, dbuf, rbuf, iD0, iD1, iS0, iS1, g0, g1, g2, g3,
             pay0, pay1, pidx0, pidx1, zbuf, slab, sem):
    c = lax.axis_index("c")
    s = lax.axis_index("s")
    wid = s * NC + c
    base = wid * EPW

    def z_body(i, _):
        zbuf[pl.ds(i * 16, 16)] = jnp.zeros((16,), _f32)
        return 0
    lax.fori_loop(0, DSTRIPE // 16, z_body, 0)
    pltpu.sync_copy(zbuf, slab.at[pl.ds(s * DSTRIPE, DSTRIPE)])
    plsc.subcore_barrier()

    pltpu.sync_copy(srcp.at[pl.ds(base, EPW)], sbuf)
    pltpu.sync_copy(dstp.at[pl.ds(base, EPW)], dbuf)
    pltpu.sync_copy(relp.at[pl.ds(base, EPW)], rbuf)

    def chunk(ci, _):
        co = ci * CH

        def emit(g, e0, e1):
            d = dbuf[pl.ds(co + g * 16, 16)]
            pay0[pl.ds(g * 16, 16)] = e0
            pay1[pl.ds(g * 16, 16)] = e1
            pidx0[pl.ds(g * 16, 16)] = d
            pidx1[pl.ds(g * 16, 16)] = d + NPAD

        _expa_chunk(qtf, ktf, sbuf, dbuf, rbuf, co, iD0, iD1, iS0, iS1,
                    g0, g1, g2, g3, sem, emit)

        pltpu.sync_copy(pay0, slab.at[pidx0], add=True)
        pltpu.sync_copy(pay1, slab.at[pidx1], add=True)
        return 0

    lax.fori_loop(0, NCHUNK, chunk, 0)
    plsc.subcore_barrier()
    pltpu.sync_copy(slab.at[pl.ds(s * DSTRIPE, DSTRIPE)],
                    denout.at[pl.ds(c * DEN_W + s * DSTRIPE, DSTRIPE)])


def _k2(qtf, ktf, srcp, dstp, relp):
    mesh = plsc.VectorSubcoreMesh(core_axis_name="c", subcore_axis_name="s")
    f = pl.kernel(
        _k2_body,
        out_type=jax.ShapeDtypeStruct((NC * DEN_W,), _f32),
        mesh=mesh,
        scratch_types=[
            pltpu.VMEM((EPW,), _i32),
            pltpu.VMEM((EPW,), _i32),
            pltpu.VMEM((EPW,), _i32),
            pltpu.VMEM((CH,), _i32),
            pltpu.VMEM((CH,), _i32),
            pltpu.VMEM((CH,), _i32),
            pltpu.VMEM((CH,), _i32),
            pltpu.VMEM((CH,), _f32),
            pltpu.VMEM((CH,), _f32),
            pltpu.VMEM((CH,), _f32),
            pltpu.VMEM((CH,), _f32),
            pltpu.VMEM((CH,), _f32),
            pltpu.VMEM((CH,), _f32),
            pltpu.VMEM((CH,), _i32),
            pltpu.VMEM((CH,), _i32),
            pltpu.VMEM((DSTRIPE,), _f32),
            pltpu.VMEM_SHARED((DEN_W,), _f32),
            pltpu.SemaphoreType.DMA,
        ],
    )
    return f(qtf, ktf, srcp, dstp, relp)


def _k3_body(den_ref, out_ref):
    out_ref[...] = 1.0 / (den_ref[0] + den_ref[1] + 1e-16)


def _k3(denP):
    den2 = denP.reshape(NC, DEN_W // 128, 128)
    return pl.pallas_call(
        _k3_body,
        out_shape=jax.ShapeDtypeStruct((DEN_W // 128, 128), _f32),
    )(den2)


def _k4_body(qtf, ktf, srcp, dstp, relp, rec0, rec1, x2d, lov_hbm, sout,
             srcb, dstb, relb, at0, at1, iD0, iD1, iS0, iS1, idxR,
             g0, g1, g2, g3, r0b, r1b, idxsub,
             xbuf, pay, pidxb, zbuf, ltab, slab, sem):
    c = lax.axis_index("c")
    s = lax.axis_index("s")
    wid = s * NC + c
    base = wid * EPW

    pltpu.sync_copy(lov_hbm, ltab)

    for zr in range(16):
        for zc in range(IN_CH // 16):
            zbuf[zr, pl.ds(zc * 16, 16)] = jnp.zeros((16,), _f32)

    pltpu.sync_copy(srcp.at[pl.ds(base, EPW)], srcb)
    pltpu.sync_copy(dstp.at[pl.ds(base, EPW)], dstb)
    pltpu.sync_copy(relp.at[pl.ds(base, EPW)], relb)

    def chunkA(ci, _):
        co = ci * CH

        def idx_extra(g, d):
            idxR[pl.ds(g * 16, 16)] = d

        def extra_fire():
            return [
                pltpu.async_copy(rec0.at[idxR], r0b, sem),
                pltpu.async_copy(rec1.at[idxR], r1b, sem),
            ]

        def emit(g, e0, e1):
            r0 = r0b[pl.ds(g * 16, 16)]
            r1 = r1b[pl.ds(g * 16, 16)]
            at0[pl.ds(co + g * 16, 16)] = e0 * r0
            at1[pl.ds(co + g * 16, 16)] = e1 * r1

        _expa_chunk(qtf, ktf, srcb, dstb, relb, co, iD0, iD1, iS0, iS1,
                    g0, g1, g2, g3, sem, emit,
                    idx_extra=idx_extra, extra_fire=extra_fire)
        return 0

    lax.fori_loop(0, NCHUNK, chunkA, 0)

    def passp(p, _):
        def zrow(zi, _):
            pltpu.sync_copy(zbuf, slab.at[pl.ds(s * ZSTRIPE + zi * 16, 16)])
            return 0
        lax.fori_loop(0, ZSTRIPE // 16, zrow, 0)
        pltpu.sync_copy(zbuf.at[pl.ds(0, ZSTRIPE % 16)],
                        slab.at[pl.ds(s * ZSTRIPE + (ZSTRIPE // 16) * 16,
                                      ZSTRIPE % 16)])
        plsc.subcore_barrier()

        lov = ltab[pl.ds(p * 16, 16)]

        def sub(si, _):
            eo = si * SUB

            for g in range(SUB // 16):
                off = eo + g * 16
                d = dstb[pl.ds(off, 16)]
                dr = d - lov
                relv = relb[pl.ds(off, 16)]
                r0v = relv * (HEADS * RANGE) + dr
                dump = _full16(DUMP_ROW)
                rowv = jnp.where(dr >= 0,
                                 jnp.where(dr < RANGE, r0v, dump), dump)
                rowv1 = jnp.where(dr >= 0,
                                  jnp.where(dr < RANGE, r0v + RANGE, dump),
                                  dump)
                pidxb[pl.ds(g * 16, 16)] = rowv
                pidxb[pl.ds(SUB + g * 16, 16)] = rowv1
                idxsub[pl.ds(g * 16, 16)] = srcb[pl.ds(off, 16)]

            pltpu.async_copy(x2d.at[idxsub], xbuf, sem).wait()

            for g in range(SUB // 16):
                a0v = at0[pl.ds(eo + g * 16, 16)]
                a1v = at1[pl.ds(eo + g * 16, 16)]
                for l in range(16):
                    e = g * 16 + l
                    a0b = jnp.full((16,), a0v[l], _f32)
                    a1b = jnp.full((16,), a1v[l], _f32)
                    for cb in range(IN_CH // 16):
                        xv = xbuf[e, pl.ds(cb * 16, 16)]
                        pay[e, pl.ds(cb * 16, 16)] = xv * a0b
                        pay[SUB + e, pl.ds(cb * 16, 16)] = xv * a1b

            pltpu.sync_copy(pay, slab.at[pidxb], add=True)
            return 0

        lax.fori_loop(0, NSUBW, sub, 0)
        plsc.subcore_barrier()

        pltpu.sync_copy(slab.at[pl.ds(s * RANGE, RANGE)],
                        sout.at[pl.ds((c * GROUPS + s) * NPADN + p * RANGE,
                                      RANGE)])
        plsc.subcore_barrier()
        return 0

    lax.fori_loop(0, NPASS, passp, 0)


def _k4(qtf, ktf, srcp, dstp, relp, rec0, rec1, x):
    lov = jnp.repeat(jnp.arange(NPASS, dtype=_i32) * RANGE, 16)
    mesh = plsc.VectorSubcoreMesh(core_axis_name="c", subcore_axis_name="s")
    f = pl.kernel(
        _k4_body,
        out_type=jax.ShapeDtypeStruct((NC * GROUPS * NPADN, IN_CH), _f32),
        mesh=mesh,
        scratch_types=[
            pltpu.VMEM((EPW,), _i32),
            pltpu.VMEM((EPW,), _i32),
            pltpu.VMEM((EPW,), _i32),
            pltpu.VMEM((EPW,), _f32),
            pltpu.VMEM((EPW,), _f32),
            pltpu.VMEM((CH,), _i32),
            pltpu.VMEM((CH,), _i32),
            pltpu.VMEM((CH,), _i32),
            pltpu.VMEM((CH,), _i32),
            pltpu.VMEM((CH,), _i32),
            pltpu.VMEM((CH,), _f32),
            pltpu.VMEM((CH,), _f32),
            pltpu.VMEM((CH,), _f32),
            pltpu.VMEM((CH,), _f32),
            pltpu.VMEM((CH,), _f32),
            pltpu.VMEM((CH,), _f32),
            pltpu.VMEM((SUB,), _i32),
            pltpu.VMEM((SUB, IN_CH), _f32),
            pltpu.VMEM((2 * SUB, IN_CH), _f32),
            pltpu.VMEM((2 * SUB,), _i32),
            pltpu.VMEM((16, IN_CH), _f32),
            pltpu.VMEM((NPASS * 16,), _i32),
            pltpu.VMEM_SHARED((SLAB_ROWS, IN_CH), _f32),
            pltpu.SemaphoreType.DMA,
        ],
    )
    return f(qtf, ktf, srcp, dstp, relp, rec0, rec1, x, lov)


def _k5_body(s00, s01, s10, s11, w_ref, b_ref, out_ref):
    r = pl.program_id(1)
    a = s00[0, 0] + s10[0, 0]
    b = s01[0, 0] + s11[0, 0]
    wr = w_ref[0]
    p0 = jnp.dot(a, wr[:, :OUT_CH], preferred_element_type=_f32)
    p1 = jnp.dot(b, wr[:, OUT_CH:], preferred_element_type=_f32)
    res = jnp.concatenate([p0, p1], axis=-1)

    @pl.when(r == 0)
    def _():
        out_ref[...] = res + b_ref[...]

    @pl.when(r > 0)
    def _():
        out_ref[...] += res


def _k5(sout, weight, bias):
    tn = 1000
    grid = (N // tn, NUM_REL)

    def sspec(sc, h):
        return pl.BlockSpec((1, 1, tn, IN_CH),
                            lambda i, r, sc=sc, h=h: (sc, 2 * r + h, i, 0))

    return pl.pallas_call(
        _k5_body,
        grid=grid,
        in_specs=[
            sspec(0, 0), sspec(0, 1), sspec(1, 0), sspec(1, 1),
            pl.BlockSpec((1, IN_CH, HEADS * OUT_CH), lambda i, r: (r, 0, 0)),
            pl.BlockSpec((1, HEADS * OUT_CH), lambda i, r: (0, 0)),
        ],
        out_specs=pl.BlockSpec((tn, HEADS * OUT_CH), lambda i, r: (i, 0)),
        out_shape=jax.ShapeDtypeStruct((N, HEADS * OUT_CH), _f32),
    )(sout, sout, sout, sout, weight, bias)


def kernel(x, edge_index, edge_label, weight, q, k, bias):
    src = edge_index[0]
    dst = edge_index[1]
    srcp = jnp.concatenate([src, jnp.zeros((PADN,), _i32)])
    dstp = jnp.concatenate([dst, jnp.full((PADN,), N, _i32)])
    relp = jnp.concatenate([edge_label, jnp.zeros((PADN,), _i32)])

    qt, kt = _k1(x, weight, q, k)
    qtf = qt.reshape(N * RH)
    ktf = kt.reshape(N * RH)

    denP = _k2(qtf, ktf, srcp, dstp, relp)
    rec = _k3(denP).reshape(DEN_W)
    rec0 = rec[:NPAD]
    rec1 = rec[NPAD:]

    sout = _k4(qtf, ktf, srcp, dstp, relp, rec0, rec1, x)
    sout4 = sout.reshape(NC, GROUPS, NPADN, IN_CH)
    out = _k5(sout4, weight, bias.reshape(1, HEADS * OUT_CH))
    return out
